# baseline (speedup 1.0000x reference)
import functools

import jax
import jax.numpy as jnp
from jax import lax
from jax.experimental import pallas as pl
from jax.experimental.pallas import tpu as pltpu
from jax.experimental.pallas import tpu_sc as plsc

EPS = 1e-15


def _norm(x):
    return jnp.clip(jnp.sqrt(jnp.sum(x * x, axis=-1, keepdims=True)), EPS, None)


def _artanh(x):
    x = jnp.clip(x, -1.0 + 1e-7, 1.0 - 1e-7)
    return 0.5 * jnp.log((1.0 + x) / (1.0 - x))


def _proj(x, c):
    n = _norm(x)
    maxnorm = (1.0 - 1e-5) / jnp.sqrt(c)
    return jnp.where(n > maxnorm, x / n * maxnorm, x)


def _expmap0(u, c):
    sc = jnp.sqrt(c)
    n = _norm(u)
    return jnp.tanh(sc * n) * u / (sc * n)


def _logmap0(x, c):
    sc = jnp.sqrt(c)
    n = _norm(x)
    return _artanh(sc * n) * x / (sc * n)


def _mobius_add(x, y, c):
    x2 = jnp.sum(x * x, axis=-1, keepdims=True)
    y2 = jnp.sum(y * y, axis=-1, keepdims=True)
    xy = jnp.sum(x * y, axis=-1, keepdims=True)
    num = (1.0 + 2.0 * c * xy + c * y2) * x + (1.0 - c * x2) * y
    den = 1.0 + 2.0 * c * xy + (c ** 2) * x2 * y2
    return num / jnp.clip(den, EPS, None)


def _mobius_scalar_mul(r, x, c):
    sc = jnp.sqrt(c)
    n = _norm(x)
    return jnp.tanh(r * _artanh(sc * n)) * x / (sc * n)


def _mid_point(x, y, c):
    t = _mobius_add(-x, y, c)
    return _mobius_add(x, _mobius_scalar_mul(0.5, t, c), c)


_RB = 1000


def _pre_body(first, x_ref, w_ref, b_ref, ad_ref, as_ref, cin_ref,
              ht_ref, sd_ref, ss_ref):
    c_in = cin_ref[0]
    xin = x_ref[...]
    if first:
        x_hyp = _proj(_expmap0(xin, c_in), c_in)
    else:
        x_hyp = xin
    xt = _logmap0(x_hyp, c_in)
    h = jnp.dot(xt, w_ref[...], preferred_element_type=jnp.float32) + b_ref[...]
    h_hyp = _proj(_expmap0(h, c_in), c_in)
    ht = _logmap0(h_hyp, c_in)
    ht_ref[...] = ht
    sd_ref[...] = jnp.sum(ht * ad_ref[...], axis=-1, keepdims=True)
    ss_ref[...] = jnp.sum(ht * as_ref[...], axis=-1, keepdims=True)


def _pre_call(first, x, Wi, bi, ad, as_, cin):
    n, d = x.shape
    grid = n // _RB
    return pl.pallas_call(
        functools.partial(_pre_body, first),
        grid=(grid,),
        in_specs=[
            pl.BlockSpec((_RB, d), lambda i: (i, 0)),
            pl.BlockSpec((d, d), lambda i: (0, 0)),
            pl.BlockSpec((1, d), lambda i: (0, 0)),
            pl.BlockSpec((1, d), lambda i: (0, 0)),
            pl.BlockSpec((1, d), lambda i: (0, 0)),
            pl.BlockSpec(memory_space=pltpu.SMEM),
        ],
        out_specs=[
            pl.BlockSpec((_RB, d), lambda i: (i, 0)),
            pl.BlockSpec((_RB, 1), lambda i: (i, 0)),
            pl.BlockSpec((_RB, 1), lambda i: (i, 0)),
        ],
        out_shape=[
            jax.ShapeDtypeStruct((n, d), jnp.float32),
            jax.ShapeDtypeStruct((n, 1), jnp.float32),
            jax.ShapeDtypeStruct((n, 1), jnp.float32),
        ],
    )(x, Wi, bi, ad, as_, cin)


def _post_body(residual, sp_ref, dp_ref, x_ref, cin_ref, cout_ref, o_ref):
    c_in = cin_ref[0]
    c_out = cout_ref[0]
    supp = sp_ref[0] + sp_ref[1]
    den = jnp.sum(dp_ref[0], axis=0)
    support = supp / jnp.clip(den, EPS, None)[:, None]
    agg = _proj(_expmap0(support, c_in), c_in)
    ot = jnp.tanh(_logmap0(agg, c_in))
    xn = _proj(_expmap0(ot, c_out), c_out)
    if residual:
        xh = x_ref[...]
        x_nc = _proj(_expmap0(_logmap0(xh, c_in), c_out), c_out)
        xn = _proj(_mid_point(x_nc, xn, c_out), c_out)
    o_ref[...] = xn


def _post_call(residual, supp, den, x_hyp, cin, cout):
    n, d = x_hyp.shape
    grid = n // _RB
    return pl.pallas_call(
        functools.partial(_post_body, residual),
        grid=(grid,),
        in_specs=[
            pl.BlockSpec((2, _RB, d), lambda i: (0, i, 0)),
            pl.BlockSpec((1, 32, _RB), lambda i: (i, 0, 0)),
            pl.BlockSpec((_RB, d), lambda i: (i, 0)),
            pl.BlockSpec(memory_space=pltpu.SMEM),
            pl.BlockSpec(memory_space=pltpu.SMEM),
        ],
        out_specs=pl.BlockSpec((_RB, d), lambda i: (i, 0)),
        out_shape=jax.ShapeDtypeStruct((n, d), jnp.float32),
    )(supp, den, x_hyp, cin, cout)


_KR = 128
_NBLK = 2


def _edge_scalar_call(sd_flat, ss_flat, src, dst):
    n = sd_flat.shape[0]
    e_total = src.shape[0]
    nc, ns = 2, 16
    nw = nc * ns
    ept = e_total // nw
    mesh = plsc.VectorSubcoreMesh(core_axis_name="c", subcore_axis_name="s")

    @functools.partial(
        pl.kernel,
        mesh=mesh,
        compiler_params=pltpu.CompilerParams(needs_layout_passes=False),
        out_type=[
            jax.ShapeDtypeStruct((e_total,), jnp.float32),
            jax.ShapeDtypeStruct((nw * n,), jnp.float32),
        ],
        scratch_types=[
            pltpu.VMEM((n,), jnp.float32),
            pltpu.VMEM((n,), jnp.float32),
            pltpu.VMEM((n,), jnp.float32),
            pltpu.VMEM((ept,), jnp.int32),
            pltpu.VMEM((ept,), jnp.int32),
            pltpu.VMEM((ept,), jnp.float32),
        ],
    )
    def k(sd_h, ss_h, src_h, dst_h, w_o, den_o, sd_v, ss_v, dn_v, src_v, dst_v, w_v):
        cid = lax.axis_index("c")
        sid = lax.axis_index("s")
        wid = sid * nc + cid
        z16 = jnp.zeros((16,), jnp.float32)

        def zdn(i, carry):
            dn_v[pl.ds(i * 16, 16)] = z16
            return carry
        lax.fori_loop(0, n // 16, zdn, 0)

        pltpu.sync_copy(sd_h, sd_v)
        pltpu.sync_copy(ss_h, ss_v)
        base = wid * ept
        pltpu.sync_copy(src_h.at[pl.ds(base, ept)], src_v)
        pltpu.sync_copy(dst_h.at[pl.ds(base, ept)], dst_v)

        def step(t, carry):
            si = src_v[pl.ds(t * 16, 16)]
            di = dst_v[pl.ds(t * 16, 16)]
            e = plsc.load_gather(sd_v, [di]) + plsc.load_gather(ss_v, [si])
            e = jnp.where(e >= 0.0, e, 0.2 * e)
            w = jnp.exp(e)
            w_v[pl.ds(t * 16, 16)] = w
            plsc.addupdate_scatter(dn_v, [di], w)
            return carry
        lax.fori_loop(0, ept // 16, step, 0)

        pltpu.sync_copy(w_v, w_o.at[pl.ds(base, ept)])
        for g in range(n // _RB):
            pltpu.sync_copy(dn_v.at[pl.ds(g * _RB, _RB)],
                            den_o.at[pl.ds((g * nw + wid) * _RB, _RB)])

    return k(sd_flat, ss_flat, src, dst)


def _edge_rows_call(ht, w_hbm, src2d, dst2d):
    n, d = ht.shape
    e_pad = w_hbm.shape[0]
    nc, ns = 2, 16
    nw = nc * ns
    ept = e_pad // nw
    cpt = ept // _KR
    cpb = cpt // _NBLK
    rpt = 624
    rem = n - rpt * ns
    mesh = plsc.VectorSubcoreMesh(core_axis_name="c", subcore_axis_name="s")

    @functools.partial(
        pl.kernel,
        mesh=mesh,
        compiler_params=pltpu.CompilerParams(needs_layout_passes=False),
        out_type=jax.ShapeDtypeStruct((nc * n, d), jnp.float32),
        scratch_types=[
            pltpu.VMEM((cpb, _KR), jnp.int32),
            pltpu.VMEM((cpb, _KR), jnp.int32),
            pltpu.VMEM((cpb * _KR,), jnp.float32),
            [pltpu.VMEM((_KR, d), jnp.float32) for _ in range(2)],
            pltpu.VMEM_SHARED((n, d), jnp.float32),
            [pltpu.SemaphoreType.DMA for _ in range(2)],
        ],
    )
    def k(ht_h, w_h, src_h, dst_h, supp_o,
          srcB, dstB, wB, rows, supp_sh, sems):
        cid = lax.axis_index("c")
        sid = lax.axis_index("s")
        wid = sid * nc + cid
        z16 = jnp.zeros((16,), jnp.float32)

        def zrow(i, carry):
            for q in range(d // 16):
                rows[0][i, pl.ds(q * 16, 16)] = z16
            return carry
        lax.fori_loop(0, _KR, zrow, 0)

        base_row = sid * rpt
        for off in range(0, 512, 128):
            pltpu.sync_copy(rows[0].at[pl.ds(0, 128)],
                            supp_sh.at[pl.ds(base_row + off, 128)])
        pltpu.sync_copy(rows[0].at[pl.ds(0, rpt - 512)],
                        supp_sh.at[pl.ds(base_row + 512, rpt - 512)])

        @pl.when(sid == ns - 1)
        def _():
            pltpu.sync_copy(rows[0].at[pl.ds(0, rem)],
                            supp_sh.at[pl.ds(ns * rpt, rem)])
        plsc.subcore_barrier()

        def fetch(c, b):
            pltpu.async_copy(ht_h.at[srcB.at[c]], rows[b], sems[b])

        def consume(c, b):
            pltpu.make_async_copy(ht_h.at[pl.ds(0, _KR)], rows[b],
                                  sems[b]).wait()
            wbase = c * _KR

            def rowmul(r4, c2):
                for v in range(4):
                    r = r4 * 4 + v
                    wv = plsc.load_gather(
                        wB, [lax.broadcast(wbase + r, (16,))])
                    for q in range(d // 16):
                        rows[b][r, pl.ds(q * 16, 16)] = (
                            rows[b][r, pl.ds(q * 16, 16)] * wv)
                return c2
            lax.fori_loop(0, _KR // 4, rowmul, 0)
            pltpu.sync_copy(rows[b], supp_sh.at[dstB.at[c]], add=True)

        for blk in range(_NBLK):
            crow = wid * cpt + blk * cpb
            pltpu.sync_copy(src_h.at[pl.ds(crow, cpb)], srcB)
            pltpu.sync_copy(dst_h.at[pl.ds(crow, cpb)], dstB)
            pltpu.sync_copy(w_h.at[pl.ds(crow * _KR, cpb * _KR)], wB)

            fetch(0, 0)

            def pair(p, carry):
                fetch(2 * p + 1, 1)
                consume(2 * p, 0)
                fetch(2 * p + 2, 0)
                consume(2 * p + 1, 1)
                return carry
            lax.fori_loop(0, cpb // 2 - 1, pair, 0)
            fetch(cpb - 1, 1)
            consume(cpb - 2, 0)
            consume(cpb - 1, 1)
        plsc.subcore_barrier()

        out_base = cid * n + base_row
        pltpu.sync_copy(supp_sh.at[pl.ds(base_row, rpt)],
                        supp_o.at[pl.ds(out_base, rpt)])

        @pl.when(sid == ns - 1)
        def _():
            pltpu.sync_copy(supp_sh.at[pl.ds(ns * rpt, rem)],
                            supp_o.at[pl.ds(cid * n + ns * rpt, rem)])

    return k(ht, w_hbm, src2d, dst2d)


def kernel(x, W, b, att, curv, edge_index, orders):
    n, d = x.shape
    n_layers = W.shape[0]
    src = edge_index[0]
    dst = edge_index[1]
    e = src.shape[0]
    e_pad = 32 * 80 * _KR
    zpad_i = jnp.zeros((e_pad - e,), jnp.int32)
    src2d = jnp.concatenate([src, zpad_i]).reshape(-1, _KR)
    dst2d = jnp.concatenate([dst, zpad_i]).reshape(-1, _KR)
    wpad = jnp.zeros((e_pad - e,), jnp.float32)
    x_hyp = x
    for i in range(n_layers):
        cin = curv[i].reshape(1)
        cout = curv[i + 1].reshape(1)
        ad = att[i, :d].reshape(1, d)
        as_ = att[i, d:].reshape(1, d)
        bi = b[i].reshape(1, d)
        ht, sd, ss = _pre_call(i == 0, x_hyp, W[i], bi, ad, as_, cin)
        w_e, den_f = _edge_scalar_call(sd.reshape(-1), ss.reshape(-1), src, dst)
        w_p = jnp.concatenate([w_e, wpad])
        supp_f = _edge_rows_call(ht, w_p, src2d, dst2d)
        supp = supp_f.reshape(2, n, d)
        den = den_f.reshape(n // _RB, 32, _RB)
        x_hyp = _post_call(i >= 1, supp, den, x_hyp, cin, cout)
    return x_hyp


# speedup vs baseline: 1.3244x; 1.3244x over previous
import functools

import jax
import jax.numpy as jnp
from jax import lax
from jax.experimental import pallas as pl
from jax.experimental.pallas import tpu as pltpu
from jax.experimental.pallas import tpu_sc as plsc

EPS = 1e-15


def _norm(x):
    return jnp.clip(jnp.sqrt(jnp.sum(x * x, axis=-1, keepdims=True)), EPS, None)


def _artanh(x):
    x = jnp.clip(x, -1.0 + 1e-7, 1.0 - 1e-7)
    return 0.5 * jnp.log((1.0 + x) / (1.0 - x))


def _proj(x, c):
    n = _norm(x)
    maxnorm = (1.0 - 1e-5) / jnp.sqrt(c)
    return jnp.where(n > maxnorm, x / n * maxnorm, x)


def _expmap0(u, c):
    sc = jnp.sqrt(c)
    n = _norm(u)
    return jnp.tanh(sc * n) * u / (sc * n)


def _logmap0(x, c):
    sc = jnp.sqrt(c)
    n = _norm(x)
    return _artanh(sc * n) * x / (sc * n)


def _mobius_add(x, y, c):
    x2 = jnp.sum(x * x, axis=-1, keepdims=True)
    y2 = jnp.sum(y * y, axis=-1, keepdims=True)
    xy = jnp.sum(x * y, axis=-1, keepdims=True)
    num = (1.0 + 2.0 * c * xy + c * y2) * x + (1.0 - c * x2) * y
    den = 1.0 + 2.0 * c * xy + (c ** 2) * x2 * y2
    return num / jnp.clip(den, EPS, None)


def _mobius_scalar_mul(r, x, c):
    sc = jnp.sqrt(c)
    n = _norm(x)
    return jnp.tanh(r * _artanh(sc * n)) * x / (sc * n)


def _mid_point(x, y, c):
    t = _mobius_add(-x, y, c)
    return _mobius_add(x, _mobius_scalar_mul(0.5, t, c), c)


_RB = 1000


def _pre_body(first, x_ref, w_ref, b_ref, ad_ref, as_ref, cin_ref,
              ht_ref, sd_ref, ss_ref):
    c_in = cin_ref[0]
    xin = x_ref[...]
    if first:
        x_hyp = _proj(_expmap0(xin, c_in), c_in)
    else:
        x_hyp = xin
    xt = _logmap0(x_hyp, c_in)
    h = jnp.dot(xt, w_ref[...], preferred_element_type=jnp.float32) + b_ref[...]
    h_hyp = _proj(_expmap0(h, c_in), c_in)
    ht = _logmap0(h_hyp, c_in)
    ht_ref[...] = ht
    sd_ref[...] = jnp.sum(ht * ad_ref[...], axis=-1, keepdims=True)
    ss_ref[...] = jnp.sum(ht * as_ref[...], axis=-1, keepdims=True)


def _pre_call(first, x, Wi, bi, ad, as_, cin):
    n, d = x.shape
    grid = n // _RB
    return pl.pallas_call(
        functools.partial(_pre_body, first),
        grid=(grid,),
        in_specs=[
            pl.BlockSpec((_RB, d), lambda i: (i, 0)),
            pl.BlockSpec((d, d), lambda i: (0, 0)),
            pl.BlockSpec((1, d), lambda i: (0, 0)),
            pl.BlockSpec((1, d), lambda i: (0, 0)),
            pl.BlockSpec((1, d), lambda i: (0, 0)),
            pl.BlockSpec(memory_space=pltpu.SMEM),
        ],
        out_specs=[
            pl.BlockSpec((_RB, d), lambda i: (i, 0)),
            pl.BlockSpec((_RB, 1), lambda i: (i, 0)),
            pl.BlockSpec((_RB, 1), lambda i: (i, 0)),
        ],
        out_shape=[
            jax.ShapeDtypeStruct((n, d), jnp.float32),
            jax.ShapeDtypeStruct((n, 1), jnp.float32),
            jax.ShapeDtypeStruct((n, 1), jnp.float32),
        ],
    )(x, Wi, bi, ad, as_, cin)


def _post_body(residual, sp_ref, dp_ref, x_ref, cin_ref, cout_ref, o_ref):
    c_in = cin_ref[0]
    c_out = cout_ref[0]
    supp = sp_ref[0] + sp_ref[1]
    den = jnp.sum(dp_ref[0], axis=0)
    support = supp / jnp.clip(den, EPS, None)[:, None]
    agg = _proj(_expmap0(support, c_in), c_in)
    ot = jnp.tanh(_logmap0(agg, c_in))
    xn = _proj(_expmap0(ot, c_out), c_out)
    if residual:
        xh = x_ref[...]
        x_nc = _proj(_expmap0(_logmap0(xh, c_in), c_out), c_out)
        xn = _proj(_mid_point(x_nc, xn, c_out), c_out)
    o_ref[...] = xn


def _post_call(residual, supp, den, x_hyp, cin, cout):
    n, d = x_hyp.shape
    grid = n // _RB
    return pl.pallas_call(
        functools.partial(_post_body, residual),
        grid=(grid,),
        in_specs=[
            pl.BlockSpec((2, _RB, d), lambda i: (0, i, 0)),
            pl.BlockSpec((1, 32, _RB), lambda i: (i, 0, 0)),
            pl.BlockSpec((_RB, d), lambda i: (i, 0)),
            pl.BlockSpec(memory_space=pltpu.SMEM),
            pl.BlockSpec(memory_space=pltpu.SMEM),
        ],
        out_specs=pl.BlockSpec((_RB, d), lambda i: (i, 0)),
        out_shape=jax.ShapeDtypeStruct((n, d), jnp.float32),
    )(supp, den, x_hyp, cin, cout)


_K = 256
_SUB = _K // 128


def _edge_scalar_call(sd_flat, ss_flat, src, dst):
    n = sd_flat.shape[0]
    e_total = src.shape[0]
    nc, ns = 2, 16
    nw = nc * ns
    ept = e_total // nw
    mesh = plsc.VectorSubcoreMesh(core_axis_name="c", subcore_axis_name="s")

    @functools.partial(
        pl.kernel,
        mesh=mesh,
        compiler_params=pltpu.CompilerParams(needs_layout_passes=False),
        out_type=[
            jax.ShapeDtypeStruct((e_total,), jnp.float32),
            jax.ShapeDtypeStruct((nw * n,), jnp.float32),
        ],
        scratch_types=[
            pltpu.VMEM((n,), jnp.float32),
            pltpu.VMEM((n,), jnp.float32),
            pltpu.VMEM((n,), jnp.float32),
            pltpu.VMEM((ept,), jnp.int32),
            pltpu.VMEM((ept,), jnp.int32),
            pltpu.VMEM((ept,), jnp.float32),
        ],
    )
    def k(sd_h, ss_h, src_h, dst_h, w_o, den_o, sd_v, ss_v, dn_v, src_v, dst_v, w_v):
        cid = lax.axis_index("c")
        sid = lax.axis_index("s")
        wid = sid * nc + cid
        z16 = jnp.zeros((16,), jnp.float32)

        def zdn(i, carry):
            dn_v[pl.ds(i * 16, 16)] = z16
            return carry
        lax.fori_loop(0, n // 16, zdn, 0)

        pltpu.sync_copy(sd_h, sd_v)
        pltpu.sync_copy(ss_h, ss_v)
        base = wid * ept
        pltpu.sync_copy(src_h.at[pl.ds(base, ept)], src_v)
        pltpu.sync_copy(dst_h.at[pl.ds(base, ept)], dst_v)

        def step(t, carry):
            si = src_v[pl.ds(t * 16, 16)]
            di = dst_v[pl.ds(t * 16, 16)]
            e = plsc.load_gather(sd_v, [di]) + plsc.load_gather(ss_v, [si])
            e = jnp.where(e >= 0.0, e, 0.2 * e)
            w = jnp.exp(e)
            w_v[pl.ds(t * 16, 16)] = w
            plsc.addupdate_scatter(dn_v, [di], w)
            return carry
        lax.fori_loop(0, ept // 16, step, 0)

        pltpu.sync_copy(w_v, w_o.at[pl.ds(base, ept)])
        for g in range(n // _RB):
            pltpu.sync_copy(dn_v.at[pl.ds(g * _RB, _RB)],
                            den_o.at[pl.ds((g * nw + wid) * _RB, _RB)])

    return k(sd_flat, ss_flat, src, dst)


def _edge_rows_call(ht, w_hbm, src, dst):
    n, d = ht.shape
    e_total = src.shape[0]
    n_chunks = e_total // _K
    nc, ns = 2, 16
    nw = nc * ns
    rpt = 624
    rem = n - rpt * ns
    mesh = plsc.VectorSubcoreMesh(core_axis_name="c", subcore_axis_name="s")

    @functools.partial(
        pl.kernel,
        mesh=mesh,
        compiler_params=pltpu.CompilerParams(needs_layout_passes=False),
        out_type=jax.ShapeDtypeStruct((nc * n, d), jnp.float32),
        scratch_types=[
            [pltpu.VMEM((128,), jnp.int32) for _ in range(_SUB)],
            [pltpu.VMEM((128,), jnp.int32) for _ in range(_SUB)],
            pltpu.VMEM((_K,), jnp.float32),
            pltpu.VMEM((_K, d), jnp.float32),
            pltpu.VMEM_SHARED((n, d), jnp.float32),
            pltpu.SemaphoreType.DMA,
        ],
    )
    def k(ht_h, w_h, src_h, dst_h, supp_o,
          srcb, dstb, w_v, rows, supp_sh, sem):
        cid = lax.axis_index("c")
        sid = lax.axis_index("s")
        wid = sid * nc + cid
        z16 = jnp.zeros((16,), jnp.float32)

        def zrow(i, carry):
            for q in range(d // 16):
                rows[i, pl.ds(q * 16, 16)] = z16
            return carry
        lax.fori_loop(0, _K, zrow, 0)

        base_row = sid * rpt
        pltpu.sync_copy(rows.at[pl.ds(0, 256)], supp_sh.at[pl.ds(base_row, 256)])
        pltpu.sync_copy(rows.at[pl.ds(0, 256)],
                        supp_sh.at[pl.ds(base_row + 256, 256)])
        pltpu.sync_copy(rows.at[pl.ds(0, rpt - 512)],
                        supp_sh.at[pl.ds(base_row + 512, rpt - 512)])

        @pl.when(sid == ns - 1)
        def _():
            pltpu.sync_copy(rows.at[pl.ds(0, rem)],
                            supp_sh.at[pl.ds(ns * rpt, rem)])
        plsc.subcore_barrier()

        n_mine = (n_chunks - wid + nw - 1) // nw

        def chunk(j, carry):
            g = wid + j * nw
            for u in range(_SUB):
                pltpu.sync_copy(src_h.at[pl.ds(g * _K + u * 128, 128)], srcb[u])
                pltpu.sync_copy(dst_h.at[pl.ds(g * _K + u * 128, 128)], dstb[u])
            pltpu.sync_copy(w_h.at[pl.ds(g * _K, _K)], w_v)
            descs = [
                pltpu.async_copy(ht_h.at[srcb[u]],
                                 rows.at[pl.ds(u * 128, 128)], sem)
                for u in range(_SUB)
            ]
            for dd in descs:
                dd.wait()

            def rowmul(r, c2):
                wv = plsc.load_gather(w_v, [lax.broadcast(r, (16,))])
                for q in range(d // 16):
                    rows[r, pl.ds(q * 16, 16)] = rows[r, pl.ds(q * 16, 16)] * wv
                return c2
            lax.fori_loop(0, _K, rowmul, 0)
            for u in range(_SUB):
                pltpu.sync_copy(rows.at[pl.ds(u * 128, 128)],
                                supp_sh.at[dstb[u]], add=True)
            return carry
        lax.fori_loop(0, n_mine, chunk, 0)
        plsc.subcore_barrier()

        out_base = cid * n + base_row
        pltpu.sync_copy(supp_sh.at[pl.ds(base_row, rpt)],
                        supp_o.at[pl.ds(out_base, rpt)])

        @pl.when(sid == ns - 1)
        def _():
            pltpu.sync_copy(supp_sh.at[pl.ds(ns * rpt, rem)],
                            supp_o.at[pl.ds(cid * n + ns * rpt, rem)])

    return k(ht, w_hbm, src, dst)


def kernel(x, W, b, att, curv, edge_index, orders):
    n, d = x.shape
    n_layers = W.shape[0]
    src = edge_index[0]
    dst = edge_index[1]
    x_hyp = x
    for i in range(n_layers):
        cin = curv[i].reshape(1)
        cout = curv[i + 1].reshape(1)
        ad = att[i, :d].reshape(1, d)
        as_ = att[i, d:].reshape(1, d)
        bi = b[i].reshape(1, d)
        ht, sd, ss = _pre_call(i == 0, x_hyp, W[i], bi, ad, as_, cin)
        w_e, den_f = _edge_scalar_call(sd.reshape(-1), ss.reshape(-1), src, dst)
        supp_f = _edge_rows_call(ht, w_e, src, dst)
        supp = supp_f.reshape(2, n, d)
        den = den_f.reshape(n // _RB, 32, _RB)
        x_hyp = _post_call(i >= 1, supp, den, x_hyp, cin, cout)
    return x_hyp


# speedup vs baseline: 1.4874x; 1.1231x over previous
import functools

import jax
import jax.numpy as jnp
from jax import lax
from jax.experimental import pallas as pl
from jax.experimental.pallas import tpu as pltpu
from jax.experimental.pallas import tpu_sc as plsc

EPS = 1e-15


def _norm(x):
    return jnp.clip(jnp.sqrt(jnp.sum(x * x, axis=-1, keepdims=True)), EPS, None)


def _artanh(x):
    x = jnp.clip(x, -1.0 + 1e-7, 1.0 - 1e-7)
    return 0.5 * jnp.log((1.0 + x) / (1.0 - x))


def _proj(x, c):
    n = _norm(x)
    maxnorm = (1.0 - 1e-5) / jnp.sqrt(c)
    return jnp.where(n > maxnorm, x / n * maxnorm, x)


def _expmap0(u, c):
    sc = jnp.sqrt(c)
    n = _norm(u)
    return jnp.tanh(sc * n) * u / (sc * n)


def _logmap0(x, c):
    sc = jnp.sqrt(c)
    n = _norm(x)
    return _artanh(sc * n) * x / (sc * n)


def _mobius_add(x, y, c):
    x2 = jnp.sum(x * x, axis=-1, keepdims=True)
    y2 = jnp.sum(y * y, axis=-1, keepdims=True)
    xy = jnp.sum(x * y, axis=-1, keepdims=True)
    num = (1.0 + 2.0 * c * xy + c * y2) * x + (1.0 - c * x2) * y
    den = 1.0 + 2.0 * c * xy + (c ** 2) * x2 * y2
    return num / jnp.clip(den, EPS, None)


def _mobius_scalar_mul(r, x, c):
    sc = jnp.sqrt(c)
    n = _norm(x)
    return jnp.tanh(r * _artanh(sc * n)) * x / (sc * n)


def _mid_point(x, y, c):
    t = _mobius_add(-x, y, c)
    return _mobius_add(x, _mobius_scalar_mul(0.5, t, c), c)


_RB = 1000


def _pre_body(first, x_ref, w_ref, b_ref, ad_ref, as_ref, cin_ref,
              ht_ref, sd_ref, ss_ref):
    c_in = cin_ref[0]
    xin = x_ref[...]
    if first:
        x_hyp = _proj(_expmap0(xin, c_in), c_in)
    else:
        x_hyp = xin
    xt = _logmap0(x_hyp, c_in)
    h = jnp.dot(xt, w_ref[...], preferred_element_type=jnp.float32) + b_ref[...]
    h_hyp = _proj(_expmap0(h, c_in), c_in)
    ht = _logmap0(h_hyp, c_in)
    ht_ref[...] = ht
    sd_ref[...] = jnp.sum(ht * ad_ref[...], axis=-1, keepdims=True)
    ss_ref[...] = jnp.sum(ht * as_ref[...], axis=-1, keepdims=True)


def _pre_call(first, x, Wi, bi, ad, as_, cin):
    n, d = x.shape
    grid = n // _RB
    return pl.pallas_call(
        functools.partial(_pre_body, first),
        grid=(grid,),
        in_specs=[
            pl.BlockSpec((_RB, d), lambda i: (i, 0)),
            pl.BlockSpec((d, d), lambda i: (0, 0)),
            pl.BlockSpec((1, d), lambda i: (0, 0)),
            pl.BlockSpec((1, d), lambda i: (0, 0)),
            pl.BlockSpec((1, d), lambda i: (0, 0)),
            pl.BlockSpec(memory_space=pltpu.SMEM),
        ],
        out_specs=[
            pl.BlockSpec((_RB, d), lambda i: (i, 0)),
            pl.BlockSpec((_RB, 1), lambda i: (i, 0)),
            pl.BlockSpec((_RB, 1), lambda i: (i, 0)),
        ],
        out_shape=[
            jax.ShapeDtypeStruct((n, d), jnp.float32),
            jax.ShapeDtypeStruct((n, 1), jnp.float32),
            jax.ShapeDtypeStruct((n, 1), jnp.float32),
        ],
    )(x, Wi, bi, ad, as_, cin)


def _post_body(residual, sp_ref, dp_ref, x_ref, cin_ref, cout_ref, o_ref):
    c_in = cin_ref[0]
    c_out = cout_ref[0]
    supp = sp_ref[0] + sp_ref[1]
    den = jnp.sum(dp_ref[0], axis=0)
    support = supp / jnp.clip(den, EPS, None)[:, None]
    agg = _proj(_expmap0(support, c_in), c_in)
    ot = jnp.tanh(_logmap0(agg, c_in))
    xn = _proj(_expmap0(ot, c_out), c_out)
    if residual:
        xh = x_ref[...]
        x_nc = _proj(_expmap0(_logmap0(xh, c_in), c_out), c_out)
        xn = _proj(_mid_point(x_nc, xn, c_out), c_out)
    o_ref[...] = xn


def _post_call(residual, supp, den, x_hyp, cin, cout):
    n, d = x_hyp.shape
    grid = n // _RB
    return pl.pallas_call(
        functools.partial(_post_body, residual),
        grid=(grid,),
        in_specs=[
            pl.BlockSpec((2, _RB, d), lambda i: (0, i, 0)),
            pl.BlockSpec((1, 32, _RB), lambda i: (i, 0, 0)),
            pl.BlockSpec((_RB, d), lambda i: (i, 0)),
            pl.BlockSpec(memory_space=pltpu.SMEM),
            pl.BlockSpec(memory_space=pltpu.SMEM),
        ],
        out_specs=pl.BlockSpec((_RB, d), lambda i: (i, 0)),
        out_shape=jax.ShapeDtypeStruct((n, d), jnp.float32),
    )(supp, den, x_hyp, cin, cout)


_K = 256
_SUB = _K // 128


def _edge_scalar_call(sd_flat, ss_flat, src, dst):
    n = sd_flat.shape[0]
    e_total = src.shape[0]
    nc, ns = 2, 16
    nw = nc * ns
    ept = e_total // nw
    mesh = plsc.VectorSubcoreMesh(core_axis_name="c", subcore_axis_name="s")

    @functools.partial(
        pl.kernel,
        mesh=mesh,
        compiler_params=pltpu.CompilerParams(needs_layout_passes=False),
        out_type=[
            jax.ShapeDtypeStruct((e_total,), jnp.float32),
            jax.ShapeDtypeStruct((nw * n,), jnp.float32),
        ],
        scratch_types=[
            pltpu.VMEM((n,), jnp.float32),
            pltpu.VMEM((n,), jnp.float32),
            pltpu.VMEM((n,), jnp.float32),
            pltpu.VMEM((ept,), jnp.int32),
            pltpu.VMEM((ept,), jnp.int32),
            pltpu.VMEM((ept,), jnp.float32),
        ],
    )
    def k(sd_h, ss_h, src_h, dst_h, w_o, den_o, sd_v, ss_v, dn_v, src_v, dst_v, w_v):
        cid = lax.axis_index("c")
        sid = lax.axis_index("s")
        wid = sid * nc + cid
        z16 = jnp.zeros((16,), jnp.float32)

        def zdn(i, carry):
            dn_v[pl.ds(i * 16, 16)] = z16
            return carry
        lax.fori_loop(0, n // 16, zdn, 0)

        pltpu.sync_copy(sd_h, sd_v)
        pltpu.sync_copy(ss_h, ss_v)
        base = wid * ept
        pltpu.sync_copy(src_h.at[pl.ds(base, ept)], src_v)
        pltpu.sync_copy(dst_h.at[pl.ds(base, ept)], dst_v)

        def step(t, carry):
            si = src_v[pl.ds(t * 16, 16)]
            di = dst_v[pl.ds(t * 16, 16)]
            e = plsc.load_gather(sd_v, [di]) + plsc.load_gather(ss_v, [si])
            e = jnp.where(e >= 0.0, e, 0.2 * e)
            w = jnp.exp(e)
            w_v[pl.ds(t * 16, 16)] = w
            plsc.addupdate_scatter(dn_v, [di], w)
            return carry
        lax.fori_loop(0, ept // 16, step, 0)

        pltpu.sync_copy(w_v, w_o.at[pl.ds(base, ept)])
        for g in range(n // _RB):
            pltpu.sync_copy(dn_v.at[pl.ds(g * _RB, _RB)],
                            den_o.at[pl.ds((g * nw + wid) * _RB, _RB)])

    return k(sd_flat, ss_flat, src, dst)


def _edge_rows_call(ht, w_hbm, src, dst):
    n, d = ht.shape
    e_total = src.shape[0]
    n_chunks = e_total // _K
    nc, ns = 2, 16
    nw = nc * ns
    rpt = 624
    rem = n - rpt * ns
    mesh = plsc.VectorSubcoreMesh(core_axis_name="c", subcore_axis_name="s")

    @functools.partial(
        pl.kernel,
        mesh=mesh,
        compiler_params=pltpu.CompilerParams(needs_layout_passes=False),
        out_type=jax.ShapeDtypeStruct((nc * n, d), jnp.float32),
        scratch_types=[
            [pltpu.VMEM((128,), jnp.int32) for _ in range(_SUB)],
            [[pltpu.VMEM((128,), jnp.int32) for _ in range(_SUB)]
             for _ in range(2)],
            pltpu.VMEM((_K,), jnp.float32),
            pltpu.VMEM((_K, d), jnp.float32),
            pltpu.VMEM_SHARED((n, d), jnp.float32),
            pltpu.SemaphoreType.DMA,
            pltpu.SemaphoreType.DMA,
        ],
    )
    def k(ht_h, w_h, src_h, dst_h, supp_o,
          srcb, dstb, w_v, rows, supp_sh, sem, sem_s):
        cid = lax.axis_index("c")
        sid = lax.axis_index("s")
        wid = sid * nc + cid
        z16 = jnp.zeros((16,), jnp.float32)

        def zrow(i, carry):
            for q in range(d // 16):
                rows[i, pl.ds(q * 16, 16)] = z16
            return carry
        lax.fori_loop(0, _K, zrow, 0)

        base_row = sid * rpt
        pltpu.sync_copy(rows.at[pl.ds(0, 256)], supp_sh.at[pl.ds(base_row, 256)])
        pltpu.sync_copy(rows.at[pl.ds(0, 256)],
                        supp_sh.at[pl.ds(base_row + 256, 256)])
        pltpu.sync_copy(rows.at[pl.ds(0, rpt - 512)],
                        supp_sh.at[pl.ds(base_row + 512, rpt - 512)])

        @pl.when(sid == ns - 1)
        def _():
            pltpu.sync_copy(rows.at[pl.ds(0, rem)],
                            supp_sh.at[pl.ds(ns * rpt, rem)])
        plsc.subcore_barrier()

        n_mine = (n_chunks - wid + nw - 1) // nw

        def drain_scatter():
            for u in range(_SUB):
                pltpu.make_async_copy(ht_h.at[pl.ds(0, 128)],
                                      rows.at[pl.ds(u * 128, 128)],
                                      sem_s).wait()

        def do_chunk(j, db):
            g = wid + j * nw
            for u in range(_SUB):
                pltpu.sync_copy(src_h.at[pl.ds(g * _K + u * 128, 128)], srcb[u])
                pltpu.sync_copy(dst_h.at[pl.ds(g * _K + u * 128, 128)], db[u])
            pltpu.sync_copy(w_h.at[pl.ds(g * _K, _K)], w_v)

            @pl.when(j > 0)
            def _():
                drain_scatter()
            descs = [
                pltpu.async_copy(ht_h.at[srcb[u]],
                                 rows.at[pl.ds(u * 128, 128)], sem)
                for u in range(_SUB)
            ]
            for dd in descs:
                dd.wait()

            def rowmul(r, c2):
                wv = plsc.load_gather(w_v, [lax.broadcast(r, (16,))])
                for q in range(d // 16):
                    rows[r, pl.ds(q * 16, 16)] = rows[r, pl.ds(q * 16, 16)] * wv
                return c2
            lax.fori_loop(0, _K, rowmul, 0)
            for u in range(_SUB):
                pltpu.async_copy(rows.at[pl.ds(u * 128, 128)],
                                 supp_sh.at[db[u]], sem_s, add=True)

        def chunk(j, carry):
            @pl.when(j % 2 == 0)
            def _():
                do_chunk(j, dstb[0])

            @pl.when(j % 2 == 1)
            def _():
                do_chunk(j, dstb[1])
            return carry
        lax.fori_loop(0, n_mine, chunk, 0)
        drain_scatter()
        plsc.subcore_barrier()

        out_base = cid * n + base_row
        pltpu.sync_copy(supp_sh.at[pl.ds(base_row, rpt)],
                        supp_o.at[pl.ds(out_base, rpt)])

        @pl.when(sid == ns - 1)
        def _():
            pltpu.sync_copy(supp_sh.at[pl.ds(ns * rpt, rem)],
                            supp_o.at[pl.ds(cid * n + ns * rpt, rem)])

    return k(ht, w_hbm, src, dst)


def kernel(x, W, b, att, curv, edge_index, orders):
    n, d = x.shape
    n_layers = W.shape[0]
    src = edge_index[0]
    dst = edge_index[1]
    x_hyp = x
    for i in range(n_layers):
        cin = curv[i].reshape(1)
        cout = curv[i + 1].reshape(1)
        ad = att[i, :d].reshape(1, d)
        as_ = att[i, d:].reshape(1, d)
        bi = b[i].reshape(1, d)
        ht, sd, ss = _pre_call(i == 0, x_hyp, W[i], bi, ad, as_, cin)
        w_e, den_f = _edge_scalar_call(sd.reshape(-1), ss.reshape(-1), src, dst)
        supp_f = _edge_rows_call(ht, w_e, src, dst)
        supp = supp_f.reshape(2, n, d)
        den = den_f.reshape(n // _RB, 32, _RB)
        x_hyp = _post_call(i >= 1, supp, den, x_hyp, cin, cout)
    return x_hyp


# speedup vs baseline: 1.6780x; 1.1282x over previous
import functools

import jax
import jax.numpy as jnp
from jax import lax
from jax.experimental import pallas as pl
from jax.experimental.pallas import tpu as pltpu
from jax.experimental.pallas import tpu_sc as plsc

EPS = 1e-15


def _norm(x):
    return jnp.clip(jnp.sqrt(jnp.sum(x * x, axis=-1, keepdims=True)), EPS, None)


def _artanh(x):
    x = jnp.clip(x, -1.0 + 1e-7, 1.0 - 1e-7)
    return 0.5 * jnp.log((1.0 + x) / (1.0 - x))


def _proj(x, c):
    n = _norm(x)
    maxnorm = (1.0 - 1e-5) / jnp.sqrt(c)
    return jnp.where(n > maxnorm, x / n * maxnorm, x)


def _expmap0(u, c):
    sc = jnp.sqrt(c)
    n = _norm(u)
    return jnp.tanh(sc * n) * u / (sc * n)


def _logmap0(x, c):
    sc = jnp.sqrt(c)
    n = _norm(x)
    return _artanh(sc * n) * x / (sc * n)


def _mobius_add(x, y, c):
    x2 = jnp.sum(x * x, axis=-1, keepdims=True)
    y2 = jnp.sum(y * y, axis=-1, keepdims=True)
    xy = jnp.sum(x * y, axis=-1, keepdims=True)
    num = (1.0 + 2.0 * c * xy + c * y2) * x + (1.0 - c * x2) * y
    den = 1.0 + 2.0 * c * xy + (c ** 2) * x2 * y2
    return num / jnp.clip(den, EPS, None)


def _mobius_scalar_mul(r, x, c):
    sc = jnp.sqrt(c)
    n = _norm(x)
    return jnp.tanh(r * _artanh(sc * n)) * x / (sc * n)


def _mid_point(x, y, c):
    t = _mobius_add(-x, y, c)
    return _mobius_add(x, _mobius_scalar_mul(0.5, t, c), c)


_RB = 1000


def _pre_body(first, x_ref, w_ref, b_ref, ad_ref, as_ref, cin_ref,
              ht_ref, sd_ref, ss_ref):
    c_in = cin_ref[0]
    xin = x_ref[...]
    if first:
        x_hyp = _proj(_expmap0(xin, c_in), c_in)
    else:
        x_hyp = xin
    xt = _logmap0(x_hyp, c_in)
    h = jnp.dot(xt, w_ref[...], preferred_element_type=jnp.float32) + b_ref[...]
    h_hyp = _proj(_expmap0(h, c_in), c_in)
    ht = _logmap0(h_hyp, c_in)
    ht_ref[...] = ht
    sd_ref[...] = jnp.sum(ht * ad_ref[...], axis=-1, keepdims=True)
    ss_ref[...] = jnp.sum(ht * as_ref[...], axis=-1, keepdims=True)


def _pre_call(first, x, Wi, bi, ad, as_, cin):
    n, d = x.shape
    grid = n // _RB
    return pl.pallas_call(
        functools.partial(_pre_body, first),
        grid=(grid,),
        in_specs=[
            pl.BlockSpec((_RB, d), lambda i: (i, 0)),
            pl.BlockSpec((d, d), lambda i: (0, 0)),
            pl.BlockSpec((1, d), lambda i: (0, 0)),
            pl.BlockSpec((1, d), lambda i: (0, 0)),
            pl.BlockSpec((1, d), lambda i: (0, 0)),
            pl.BlockSpec(memory_space=pltpu.SMEM),
        ],
        out_specs=[
            pl.BlockSpec((_RB, d), lambda i: (i, 0)),
            pl.BlockSpec((_RB, 1), lambda i: (i, 0)),
            pl.BlockSpec((_RB, 1), lambda i: (i, 0)),
        ],
        out_shape=[
            jax.ShapeDtypeStruct((n, d), jnp.float32),
            jax.ShapeDtypeStruct((n, 1), jnp.float32),
            jax.ShapeDtypeStruct((n, 1), jnp.float32),
        ],
    )(x, Wi, bi, ad, as_, cin)


def _post_body(residual, sp_ref, dp_ref, x_ref, cin_ref, cout_ref, o_ref):
    c_in = cin_ref[0]
    c_out = cout_ref[0]
    supp = sp_ref[0] + sp_ref[1]
    den = jnp.sum(dp_ref[0], axis=0)
    support = supp / jnp.clip(den, EPS, None)[:, None]
    agg = _proj(_expmap0(support, c_in), c_in)
    ot = jnp.tanh(_logmap0(agg, c_in))
    xn = _proj(_expmap0(ot, c_out), c_out)
    if residual:
        xh = x_ref[...]
        x_nc = _proj(_expmap0(_logmap0(xh, c_in), c_out), c_out)
        xn = _proj(_mid_point(x_nc, xn, c_out), c_out)
    o_ref[...] = xn


def _post_call(residual, supp, den, x_hyp, cin, cout):
    n, d = x_hyp.shape
    grid = n // _RB
    return pl.pallas_call(
        functools.partial(_post_body, residual),
        grid=(grid,),
        in_specs=[
            pl.BlockSpec((2, _RB, d), lambda i: (0, i, 0)),
            pl.BlockSpec((1, 32, _RB), lambda i: (i, 0, 0)),
            pl.BlockSpec((_RB, d), lambda i: (i, 0)),
            pl.BlockSpec(memory_space=pltpu.SMEM),
            pl.BlockSpec(memory_space=pltpu.SMEM),
        ],
        out_specs=pl.BlockSpec((_RB, d), lambda i: (i, 0)),
        out_shape=jax.ShapeDtypeStruct((n, d), jnp.float32),
    )(supp, den, x_hyp, cin, cout)


_K = 256
_SUB = _K // 128


def _edge_scalar_call(sd_flat, ss_flat, src, dst):
    n = sd_flat.shape[0]
    e_total = src.shape[0]
    nc, ns = 2, 16
    nw = nc * ns
    ept = e_total // nw
    mesh = plsc.VectorSubcoreMesh(core_axis_name="c", subcore_axis_name="s")

    @functools.partial(
        pl.kernel,
        mesh=mesh,
        compiler_params=pltpu.CompilerParams(needs_layout_passes=False),
        out_type=[
            jax.ShapeDtypeStruct((e_total,), jnp.float32),
            jax.ShapeDtypeStruct((nw * n,), jnp.float32),
        ],
        scratch_types=[
            pltpu.VMEM((n,), jnp.float32),
            pltpu.VMEM((n,), jnp.float32),
            pltpu.VMEM((n,), jnp.float32),
            pltpu.VMEM((ept,), jnp.int32),
            pltpu.VMEM((ept,), jnp.int32),
            pltpu.VMEM((ept,), jnp.float32),
        ],
    )
    def k(sd_h, ss_h, src_h, dst_h, w_o, den_o, sd_v, ss_v, dn_v, src_v, dst_v, w_v):
        cid = lax.axis_index("c")
        sid = lax.axis_index("s")
        wid = sid * nc + cid
        z16 = jnp.zeros((16,), jnp.float32)

        def zdn(i, carry):
            dn_v[pl.ds(i * 16, 16)] = z16
            return carry
        lax.fori_loop(0, n // 16, zdn, 0)

        pltpu.sync_copy(sd_h, sd_v)
        pltpu.sync_copy(ss_h, ss_v)
        base = wid * ept
        pltpu.sync_copy(src_h.at[pl.ds(base, ept)], src_v)
        pltpu.sync_copy(dst_h.at[pl.ds(base, ept)], dst_v)

        def step(t, carry):
            si = src_v[pl.ds(t * 16, 16)]
            di = dst_v[pl.ds(t * 16, 16)]
            e = plsc.load_gather(sd_v, [di]) + plsc.load_gather(ss_v, [si])
            e = jnp.where(e >= 0.0, e, 0.2 * e)
            w = jnp.exp(e)
            w_v[pl.ds(t * 16, 16)] = w
            plsc.addupdate_scatter(dn_v, [di], w)
            return carry
        lax.fori_loop(0, ept // 16, step, 0)

        pltpu.sync_copy(w_v, w_o.at[pl.ds(base, ept)])
        for g in range(n // _RB):
            pltpu.sync_copy(dn_v.at[pl.ds(g * _RB, _RB)],
                            den_o.at[pl.ds((g * nw + wid) * _RB, _RB)])

    return k(sd_flat, ss_flat, src, dst)


def _edge_rows_call(ht, w_hbm, src, dst):
    n, d = ht.shape
    e_total = src.shape[0]
    n_chunks = e_total // _K
    nc, ns = 2, 16
    nw = nc * ns
    rpt = 624
    rem = n - rpt * ns
    mesh = plsc.VectorSubcoreMesh(core_axis_name="c", subcore_axis_name="s")

    @functools.partial(
        pl.kernel,
        mesh=mesh,
        compiler_params=pltpu.CompilerParams(needs_layout_passes=False),
        out_type=jax.ShapeDtypeStruct((nc * n, d), jnp.float32),
        scratch_types=[
            [pltpu.VMEM((128,), jnp.int32) for _ in range(_SUB)],
            [[pltpu.VMEM((128,), jnp.int32) for _ in range(_SUB)]
             for _ in range(2)],
            pltpu.VMEM((_K,), jnp.float32),
            pltpu.VMEM((_K, d), jnp.float32),
            pltpu.VMEM_SHARED((n, d), jnp.float32),
            pltpu.SemaphoreType.DMA,
            pltpu.SemaphoreType.DMA,
        ],
    )
    def k(ht_h, w_h, src_h, dst_h, supp_o,
          srcb, dstb, w_v, rows, supp_sh, sem, sem_s):
        cid = lax.axis_index("c")
        sid = lax.axis_index("s")
        wid = sid * nc + cid
        z16 = jnp.zeros((16,), jnp.float32)

        def zrow(i, carry):
            for q in range(d // 16):
                rows[i, pl.ds(q * 16, 16)] = z16
            return carry
        lax.fori_loop(0, _K, zrow, 0)

        base_row = sid * rpt
        pltpu.sync_copy(rows.at[pl.ds(0, 256)], supp_sh.at[pl.ds(base_row, 256)])
        pltpu.sync_copy(rows.at[pl.ds(0, 256)],
                        supp_sh.at[pl.ds(base_row + 256, 256)])
        pltpu.sync_copy(rows.at[pl.ds(0, rpt - 512)],
                        supp_sh.at[pl.ds(base_row + 512, rpt - 512)])

        @pl.when(sid == ns - 1)
        def _():
            pltpu.sync_copy(rows.at[pl.ds(0, rem)],
                            supp_sh.at[pl.ds(ns * rpt, rem)])
        plsc.subcore_barrier()

        n_mine = (n_chunks - wid + nw - 1) // nw

        def drain_scatter():
            for u in range(_SUB):
                pltpu.make_async_copy(ht_h.at[pl.ds(0, 128)],
                                      rows.at[pl.ds(u * 128, 128)],
                                      sem_s).wait()

        def do_chunk(j, db):
            g = wid + j * nw
            for u in range(_SUB):
                pltpu.sync_copy(src_h.at[pl.ds(g * _K + u * 128, 128)], srcb[u])

            @pl.when(j > 0)
            def _():
                drain_scatter()
            descs = [
                pltpu.async_copy(ht_h.at[srcb[u]],
                                 rows.at[pl.ds(u * 128, 128)], sem)
                for u in range(_SUB)
            ]
            for u in range(_SUB):
                pltpu.sync_copy(dst_h.at[pl.ds(g * _K + u * 128, 128)], db[u])
            pltpu.sync_copy(w_h.at[pl.ds(g * _K, _K)], w_v)
            for dd in descs:
                dd.wait()

            def rowmul(r, c2):
                wv = plsc.load_gather(w_v, [lax.broadcast(r, (16,))])
                for q in range(d // 16):
                    rows[r, pl.ds(q * 16, 16)] = rows[r, pl.ds(q * 16, 16)] * wv
                return c2
            lax.fori_loop(0, _K, rowmul, 0)
            for u in range(_SUB):
                pltpu.async_copy(rows.at[pl.ds(u * 128, 128)],
                                 supp_sh.at[db[u]], sem_s, add=True)

        def chunk(j, carry):
            @pl.when(j % 2 == 0)
            def _():
                do_chunk(j, dstb[0])

            @pl.when(j % 2 == 1)
            def _():
                do_chunk(j, dstb[1])
            return carry
        lax.fori_loop(0, n_mine, chunk, 0)
        drain_scatter()
        plsc.subcore_barrier()

        out_base = cid * n + base_row
        pltpu.sync_copy(supp_sh.at[pl.ds(base_row, rpt)],
                        supp_o.at[pl.ds(out_base, rpt)])

        @pl.when(sid == ns - 1)
        def _():
            pltpu.sync_copy(supp_sh.at[pl.ds(ns * rpt, rem)],
                            supp_o.at[pl.ds(cid * n + ns * rpt, rem)])

    return k(ht, w_hbm, src, dst)


def kernel(x, W, b, att, curv, edge_index, orders):
    n, d = x.shape
    n_layers = W.shape[0]
    src = edge_index[0]
    dst = edge_index[1]
    x_hyp = x
    for i in range(n_layers):
        cin = curv[i].reshape(1)
        cout = curv[i + 1].reshape(1)
        ad = att[i, :d].reshape(1, d)
        as_ = att[i, d:].reshape(1, d)
        bi = b[i].reshape(1, d)
        ht, sd, ss = _pre_call(i == 0, x_hyp, W[i], bi, ad, as_, cin)
        w_e, den_f = _edge_scalar_call(sd.reshape(-1), ss.reshape(-1), src, dst)
        supp_f = _edge_rows_call(ht, w_e, src, dst)
        supp = supp_f.reshape(2, n, d)
        den = den_f.reshape(n // _RB, 32, _RB)
        x_hyp = _post_call(i >= 1, supp, den, x_hyp, cin, cout)
    return x_hyp


# speedup vs baseline: 1.7279x; 1.0297x over previous
import functools

import jax
import jax.numpy as jnp
from jax import lax
from jax.experimental import pallas as pl
from jax.experimental.pallas import tpu as pltpu
from jax.experimental.pallas import tpu_sc as plsc

EPS = 1e-15


def _norm(x):
    return jnp.clip(jnp.sqrt(jnp.sum(x * x, axis=-1, keepdims=True)), EPS, None)


def _artanh(x):
    x = jnp.clip(x, -1.0 + 1e-7, 1.0 - 1e-7)
    return 0.5 * jnp.log((1.0 + x) / (1.0 - x))


def _proj(x, c):
    n = _norm(x)
    maxnorm = (1.0 - 1e-5) / jnp.sqrt(c)
    return jnp.where(n > maxnorm, x / n * maxnorm, x)


def _expmap0(u, c):
    sc = jnp.sqrt(c)
    n = _norm(u)
    return jnp.tanh(sc * n) * u / (sc * n)


def _logmap0(x, c):
    sc = jnp.sqrt(c)
    n = _norm(x)
    return _artanh(sc * n) * x / (sc * n)


def _mobius_add(x, y, c):
    x2 = jnp.sum(x * x, axis=-1, keepdims=True)
    y2 = jnp.sum(y * y, axis=-1, keepdims=True)
    xy = jnp.sum(x * y, axis=-1, keepdims=True)
    num = (1.0 + 2.0 * c * xy + c * y2) * x + (1.0 - c * x2) * y
    den = 1.0 + 2.0 * c * xy + (c ** 2) * x2 * y2
    return num / jnp.clip(den, EPS, None)


def _mobius_scalar_mul(r, x, c):
    sc = jnp.sqrt(c)
    n = _norm(x)
    return jnp.tanh(r * _artanh(sc * n)) * x / (sc * n)


def _mid_point(x, y, c):
    t = _mobius_add(-x, y, c)
    return _mobius_add(x, _mobius_scalar_mul(0.5, t, c), c)


_RB = 1000


def _pre_body(first, x_ref, w_ref, b_ref, ad_ref, as_ref, cin_ref,
              ht_ref, sd_ref, ss_ref):
    c_in = cin_ref[0]
    xin = x_ref[...]
    if first:
        x_hyp = _proj(_expmap0(xin, c_in), c_in)
    else:
        x_hyp = xin
    xt = _logmap0(x_hyp, c_in)
    h = jnp.dot(xt, w_ref[...], preferred_element_type=jnp.float32) + b_ref[...]
    h_hyp = _proj(_expmap0(h, c_in), c_in)
    ht = _logmap0(h_hyp, c_in)
    ht_ref[...] = ht
    sd_ref[...] = jnp.sum(ht * ad_ref[...], axis=-1, keepdims=True)
    ss_ref[...] = jnp.sum(ht * as_ref[...], axis=-1, keepdims=True)


def _pre_call(first, x, Wi, bi, ad, as_, cin):
    n, d = x.shape
    grid = n // _RB
    return pl.pallas_call(
        functools.partial(_pre_body, first),
        grid=(grid,),
        in_specs=[
            pl.BlockSpec((_RB, d), lambda i: (i, 0)),
            pl.BlockSpec((d, d), lambda i: (0, 0)),
            pl.BlockSpec((1, d), lambda i: (0, 0)),
            pl.BlockSpec((1, d), lambda i: (0, 0)),
            pl.BlockSpec((1, d), lambda i: (0, 0)),
            pl.BlockSpec(memory_space=pltpu.SMEM),
        ],
        out_specs=[
            pl.BlockSpec((_RB, d), lambda i: (i, 0)),
            pl.BlockSpec((_RB, 1), lambda i: (i, 0)),
            pl.BlockSpec((_RB, 1), lambda i: (i, 0)),
        ],
        out_shape=[
            jax.ShapeDtypeStruct((n, d), jnp.float32),
            jax.ShapeDtypeStruct((n, 1), jnp.float32),
            jax.ShapeDtypeStruct((n, 1), jnp.float32),
        ],
    )(x, Wi, bi, ad, as_, cin)


def _post_body(residual, sp_ref, dp_ref, x_ref, cin_ref, cout_ref, o_ref):
    c_in = cin_ref[0]
    c_out = cout_ref[0]
    supp = sp_ref[0] + sp_ref[1]
    den = jnp.sum(dp_ref[0], axis=0)
    support = supp / jnp.clip(den, EPS, None)[:, None]
    agg = _proj(_expmap0(support, c_in), c_in)
    ot = jnp.tanh(_logmap0(agg, c_in))
    xn = _proj(_expmap0(ot, c_out), c_out)
    if residual:
        xh = x_ref[...]
        x_nc = _proj(_expmap0(_logmap0(xh, c_in), c_out), c_out)
        xn = _proj(_mid_point(x_nc, xn, c_out), c_out)
    o_ref[...] = xn


def _post_call(residual, supp, den, x_hyp, cin, cout):
    n, d = x_hyp.shape
    grid = n // _RB
    return pl.pallas_call(
        functools.partial(_post_body, residual),
        grid=(grid,),
        in_specs=[
            pl.BlockSpec((2, _RB, d), lambda i: (0, i, 0)),
            pl.BlockSpec((1, 32, _RB), lambda i: (i, 0, 0)),
            pl.BlockSpec((_RB, d), lambda i: (i, 0)),
            pl.BlockSpec(memory_space=pltpu.SMEM),
            pl.BlockSpec(memory_space=pltpu.SMEM),
        ],
        out_specs=pl.BlockSpec((_RB, d), lambda i: (i, 0)),
        out_shape=jax.ShapeDtypeStruct((n, d), jnp.float32),
    )(supp, den, x_hyp, cin, cout)


_K = 256
_SUB = _K // 128


def _edge_scalar_call(sd_flat, ss_flat, src, dst):
    n = sd_flat.shape[0]
    e_total = src.shape[0]
    nc, ns = 2, 16
    nw = nc * ns
    ept = e_total // nw
    mesh = plsc.VectorSubcoreMesh(core_axis_name="c", subcore_axis_name="s")

    @functools.partial(
        pl.kernel,
        mesh=mesh,
        compiler_params=pltpu.CompilerParams(needs_layout_passes=False),
        out_type=[
            jax.ShapeDtypeStruct((e_total,), jnp.float32),
            jax.ShapeDtypeStruct((nw * n,), jnp.float32),
        ],
        scratch_types=[
            pltpu.VMEM((n,), jnp.float32),
            pltpu.VMEM((n,), jnp.float32),
            pltpu.VMEM((n,), jnp.float32),
            pltpu.VMEM((ept,), jnp.int32),
            pltpu.VMEM((ept,), jnp.int32),
            pltpu.VMEM((ept,), jnp.float32),
        ],
    )
    def k(sd_h, ss_h, src_h, dst_h, w_o, den_o, sd_v, ss_v, dn_v, src_v, dst_v, w_v):
        cid = lax.axis_index("c")
        sid = lax.axis_index("s")
        wid = sid * nc + cid
        z16 = jnp.zeros((16,), jnp.float32)

        def zdn(i, carry):
            dn_v[pl.ds(i * 16, 16)] = z16
            return carry
        lax.fori_loop(0, n // 16, zdn, 0)

        pltpu.sync_copy(sd_h, sd_v)
        pltpu.sync_copy(ss_h, ss_v)
        base = wid * ept
        pltpu.sync_copy(src_h.at[pl.ds(base, ept)], src_v)
        pltpu.sync_copy(dst_h.at[pl.ds(base, ept)], dst_v)

        def step(t, carry):
            si = src_v[pl.ds(t * 16, 16)]
            di = dst_v[pl.ds(t * 16, 16)]
            e = plsc.load_gather(sd_v, [di]) + plsc.load_gather(ss_v, [si])
            e = jnp.where(e >= 0.0, e, 0.2 * e)
            w = jnp.exp(e)
            w_v[pl.ds(t * 16, 16)] = w
            plsc.addupdate_scatter(dn_v, [di], w)
            return carry
        lax.fori_loop(0, ept // 16, step, 0)

        pltpu.sync_copy(w_v, w_o.at[pl.ds(base, ept)])
        for g in range(n // _RB):
            pltpu.sync_copy(dn_v.at[pl.ds(g * _RB, _RB)],
                            den_o.at[pl.ds((g * nw + wid) * _RB, _RB)])

    return k(sd_flat, ss_flat, src, dst)


def _edge_rows_call(ht, w_hbm, src, dst):
    n, d = ht.shape
    e_total = src.shape[0]
    n_chunks = e_total // _K
    nc, ns = 2, 16
    nw = nc * ns
    rpt = 624
    rem = n - rpt * ns
    mesh = plsc.VectorSubcoreMesh(core_axis_name="c", subcore_axis_name="s")

    @functools.partial(
        pl.kernel,
        mesh=mesh,
        compiler_params=pltpu.CompilerParams(needs_layout_passes=False),
        out_type=jax.ShapeDtypeStruct((nc * n, d), jnp.float32),
        scratch_types=[
            [pltpu.VMEM((128,), jnp.int32) for _ in range(_SUB)],
            [[pltpu.VMEM((128,), jnp.int32) for _ in range(_SUB)]
             for _ in range(2)],
            pltpu.VMEM((_K,), jnp.float32),
            pltpu.VMEM((_K, d), jnp.float32),
            pltpu.VMEM_SHARED((n, d), jnp.float32),
            pltpu.SemaphoreType.DMA,
            pltpu.SemaphoreType.DMA,
        ],
    )
    def k(ht_h, w_h, src_h, dst_h, supp_o,
          srcb, dstb, w_v, rows, supp_sh, sem, sem_s):
        cid = lax.axis_index("c")
        sid = lax.axis_index("s")
        wid = sid * nc + cid
        z16 = jnp.zeros((16,), jnp.float32)

        def zrow(i, carry):
            for q in range(d // 16):
                rows[i, pl.ds(q * 16, 16)] = z16
            return carry
        lax.fori_loop(0, _K, zrow, 0)

        base_row = sid * rpt
        pltpu.sync_copy(rows.at[pl.ds(0, 256)], supp_sh.at[pl.ds(base_row, 256)])
        pltpu.sync_copy(rows.at[pl.ds(0, 256)],
                        supp_sh.at[pl.ds(base_row + 256, 256)])
        pltpu.sync_copy(rows.at[pl.ds(0, rpt - 512)],
                        supp_sh.at[pl.ds(base_row + 512, rpt - 512)])

        @pl.when(sid == ns - 1)
        def _():
            pltpu.sync_copy(rows.at[pl.ds(0, rem)],
                            supp_sh.at[pl.ds(ns * rpt, rem)])
        plsc.subcore_barrier()

        n_mine = (n_chunks - wid + nw - 1) // nw

        def drain_scatter():
            for u in range(_SUB):
                pltpu.make_async_copy(ht_h.at[pl.ds(0, 128)],
                                      rows.at[pl.ds(u * 128, 128)],
                                      sem_s).wait()

        def do_chunk(j, db):
            g = wid + j * nw
            for u in range(_SUB):
                pltpu.sync_copy(src_h.at[pl.ds(g * _K + u * 128, 128)], srcb[u])

            @pl.when(j > 0)
            def _():
                drain_scatter()
            descs = [
                pltpu.async_copy(ht_h.at[srcb[u]],
                                 rows.at[pl.ds(u * 128, 128)], sem)
                for u in range(_SUB)
            ]
            for u in range(_SUB):
                pltpu.sync_copy(dst_h.at[pl.ds(g * _K + u * 128, 128)], db[u])
            pltpu.sync_copy(w_h.at[pl.ds(g * _K, _K)], w_v)
            for dd in descs:
                dd.wait()

            def rowmul(r4, c2):
                for v in range(4):
                    r = r4 * 4 + v
                    wv = plsc.load_gather(w_v, [lax.broadcast(r, (16,))])
                    for q in range(d // 16):
                        rows[r, pl.ds(q * 16, 16)] = (
                            rows[r, pl.ds(q * 16, 16)] * wv)
                return c2
            lax.fori_loop(0, _K // 4, rowmul, 0)
            for u in range(_SUB):
                pltpu.async_copy(rows.at[pl.ds(u * 128, 128)],
                                 supp_sh.at[db[u]], sem_s, add=True)

        def chunk(j, carry):
            @pl.when(j % 2 == 0)
            def _():
                do_chunk(j, dstb[0])

            @pl.when(j % 2 == 1)
            def _():
                do_chunk(j, dstb[1])
            return carry
        lax.fori_loop(0, n_mine, chunk, 0)
        drain_scatter()
        plsc.subcore_barrier()

        out_base = cid * n + base_row
        pltpu.sync_copy(supp_sh.at[pl.ds(base_row, rpt)],
                        supp_o.at[pl.ds(out_base, rpt)])

        @pl.when(sid == ns - 1)
        def _():
            pltpu.sync_copy(supp_sh.at[pl.ds(ns * rpt, rem)],
                            supp_o.at[pl.ds(cid * n + ns * rpt, rem)])

    return k(ht, w_hbm, src, dst)


def kernel(x, W, b, att, curv, edge_index, orders):
    n, d = x.shape
    n_layers = W.shape[0]
    src = edge_index[0]
    dst = edge_index[1]
    x_hyp = x
    for i in range(n_layers):
        cin = curv[i].reshape(1)
        cout = curv[i + 1].reshape(1)
        ad = att[i, :d].reshape(1, d)
        as_ = att[i, d:].reshape(1, d)
        bi = b[i].reshape(1, d)
        ht, sd, ss = _pre_call(i == 0, x_hyp, W[i], bi, ad, as_, cin)
        w_e, den_f = _edge_scalar_call(sd.reshape(-1), ss.reshape(-1), src, dst)
        supp_f = _edge_rows_call(ht, w_e, src, dst)
        supp = supp_f.reshape(2, n, d)
        den = den_f.reshape(n // _RB, 32, _RB)
        x_hyp = _post_call(i >= 1, supp, den, x_hyp, cin, cout)
    return x_hyp
